# baseline (speedup 1.0000x reference)
import functools

import jax
import jax.numpy as jnp
from jax import lax
from jax.experimental import pallas as pl
from jax.experimental.pallas import tpu as pltpu
from jax.experimental.pallas import tpu_sc as plsc

TOP_K = 2
G = 256


def _router_body(x_ref, gw_ref, gb_ref, logits_ref, idx_ref, wn_ref):
    x = x_ref[...]
    logits = lax.dot_general(
        x, gw_ref[...], (((1,), (1,)), ((), ())),
        preferred_element_type=jnp.float32) + gb_ref[...]
    logits_ref[...] = logits
    m1 = jnp.max(logits, axis=-1, keepdims=True)
    a1 = jnp.argmax(logits, axis=-1)
    cols = lax.broadcasted_iota(jnp.int32, logits.shape, 1)
    logits2 = jnp.where(cols == a1[:, None], -jnp.inf, logits)
    m2 = jnp.max(logits2, axis=-1, keepdims=True)
    a2 = jnp.argmax(logits2, axis=-1)
    w1 = 1.0 / (1.0 + jnp.exp(m2 - m1))
    idx_ref[...] = jnp.concatenate(
        [a1[:, None], a2[:, None]], axis=1).astype(jnp.int32)
    wn_ref[...] = jnp.concatenate([w1, 1.0 - w1], axis=1)


def _router(xs, gate_W, gate_b):
    T, H = xs.shape
    E = gate_W.shape[0]
    TM = 1024
    return pl.pallas_call(
        _router_body,
        grid=(T // TM,),
        in_specs=[
            pl.BlockSpec((TM, H), lambda i: (i, 0)),
            pl.BlockSpec((E, H), lambda i: (0, 0)),
            pl.BlockSpec((1, E), lambda i: (0, 0)),
        ],
        out_specs=[
            pl.BlockSpec((TM, E), lambda i: (i, 0)),
            pl.BlockSpec((TM, TOP_K), lambda i: (i, 0)),
            pl.BlockSpec((TM, TOP_K), lambda i: (i, 0)),
        ],
        out_shape=[
            jax.ShapeDtypeStruct((T, E), jnp.float32),
            jax.ShapeDtypeStruct((T, TOP_K), jnp.int32),
            jax.ShapeDtypeStruct((T, TOP_K), jnp.float32),
        ],
    )(xs, gate_W, gate_b.reshape(1, E))


def _make_dispatch(T, H, P):
    info = plsc.get_sparse_core_info()
    NC, NS = info.num_cores, info.num_subcores
    NW = NC * NS
    tok_w = T // NW
    CH = 16
    n_ch = tok_w // CH
    mesh = plsc.VectorSubcoreMesh(core_axis_name="c", subcore_axis_name="s")

    @functools.partial(
        pl.kernel, mesh=mesh,
        out_type=jax.ShapeDtypeStruct((P, H), jnp.float32),
        scratch_types=[
            pltpu.VMEM((CH, H), jnp.float32),
            pltpu.VMEM((CH, H), jnp.float32),
            pltpu.VMEM((CH,), jnp.int32),
            pltpu.VMEM((CH,), jnp.int32),
            pltpu.VMEM((CH,), jnp.int32),
            pltpu.VMEM((CH,), jnp.int32),
            pltpu.SemaphoreType.DMA,
            pltpu.SemaphoreType.DMA,
            pltpu.SemaphoreType.DMA,
            pltpu.SemaphoreType.DMA,
        ],
    )
    def dispatch(x_hbm, d0_hbm, d1_hbm, a_hbm,
                 xva, xvb, i0a, i0b, i1a, i1b, s0a, s0b, s1a, s1b):
        wid = lax.axis_index("s") * NC + lax.axis_index("c")
        base = pl.multiple_of(wid * tok_w, CH)
        bufs = ((xva, i0a, i1a, s0a, s1a), (xvb, i0b, i1b, s0b, s1b))

        def _pair(j2, carry):
            for b in (0, 1):
                xv, i0_v, i1_v, sem0, sem1 = bufs[b]
                j = j2 * 2 + b

                @pl.when(j2 > 0)
                def _drain():
                    pltpu.make_async_copy(xv, a_hbm.at[i0_v], sem0).wait()
                    pltpu.make_async_copy(xv, a_hbm.at[i1_v], sem1).wait()

                bb = pl.multiple_of(base + j * CH, CH)
                pltpu.sync_copy(x_hbm.at[pl.ds(bb, CH)], xv)
                pltpu.sync_copy(d0_hbm.at[pl.ds(bb, CH)], i0_v)
                pltpu.sync_copy(d1_hbm.at[pl.ds(bb, CH)], i1_v)
                pltpu.async_copy(xv, a_hbm.at[i0_v], sem0)
                pltpu.async_copy(xv, a_hbm.at[i1_v], sem1)
            return carry

        lax.fori_loop(0, n_ch // 2, _pair, 0)
        for b in (0, 1):
            xv, i0_v, i1_v, sem0, sem1 = bufs[b]
            pltpu.make_async_copy(xv, a_hbm.at[i0_v], sem0).wait()
            pltpu.make_async_copy(xv, a_hbm.at[i1_v], sem1).wait()

    return dispatch


def _gmm_body(te_ref, a_ref, w_ref, eb_ref, y_ref):
    acc = lax.dot_general(
        a_ref[...], w_ref[0], (((1,), (1,)), ((), ())),
        preferred_element_type=jnp.float32)
    y_ref[...] = acc + eb_ref[0]


def _gmm(te, A, eW, eb3, P, H, E):
    NT = P // G
    grid_spec = pltpu.PrefetchScalarGridSpec(
        num_scalar_prefetch=1,
        grid=(NT,),
        in_specs=[
            pl.BlockSpec((G, H), lambda i, te: (i, 0)),
            pl.BlockSpec((1, H, H), lambda i, te: (te[i], 0, 0)),
            pl.BlockSpec((1, 1, H), lambda i, te: (te[i], 0, 0)),
        ],
        out_specs=pl.BlockSpec((G, H), lambda i, te: (i, 0)),
    )
    return pl.pallas_call(
        _gmm_body,
        grid_spec=grid_spec,
        out_shape=jax.ShapeDtypeStruct((P, H), jnp.float32),
    )(te, A, eW, eb3)


def _make_combine(T, H, P):
    info = plsc.get_sparse_core_info()
    NC, NS = info.num_cores, info.num_subcores
    NW = NC * NS
    tok_w = T // NW
    CH = 8
    n_ch = tok_w // CH
    mesh = plsc.VectorSubcoreMesh(core_axis_name="c", subcore_axis_name="s")

    buf = lambda *s: [pltpu.VMEM(s, jnp.float32) for _ in range(2)]

    @functools.partial(
        pl.kernel, mesh=mesh,
        out_type=jax.ShapeDtypeStruct((T, H), jnp.float32),
        scratch_types=(
            [pltpu.VMEM((CH,), jnp.int32) for _ in range(4)]
            + buf(CH, 16) + buf(CH, 16) + buf(CH, H) + buf(CH, H) + buf(CH, H)
            + [pltpu.SemaphoreType.DMA for _ in range(4)]
        ),
    )
    def combine(y_hbm, d0_hbm, d1_hbm, w0_hbm, w1_hbm, out_hbm,
                i0a, i0b, i1a, i1b, w0a, w0b, w1a, w1b,
                y0a, y0b, y1a, y1b, oa, ob, s0a, s0b, s1a, s1b):
        wid = lax.axis_index("s") * NC + lax.axis_index("c")
        base = pl.multiple_of(wid * tok_w, CH)
        ng = H // 16
        bufs = ((i0a, i1a, w0a, w1a, y0a, y1a, oa, s0a, s1a),
                (i0b, i1b, w0b, w1b, y0b, y1b, ob, s0b, s1b))

        def _load(j, b):
            i0_v, i1_v, w0_v, w1_v, y0_v, y1_v, o_v, sem0, sem1 = bufs[b]
            bb = pl.multiple_of(base + j * CH, CH)
            pltpu.sync_copy(d0_hbm.at[pl.ds(bb, CH)], i0_v)
            pltpu.sync_copy(d1_hbm.at[pl.ds(bb, CH)], i1_v)
            pltpu.sync_copy(w0_hbm.at[pl.ds(bb, CH)], w0_v)
            pltpu.sync_copy(w1_hbm.at[pl.ds(bb, CH)], w1_v)
            pltpu.async_copy(y_hbm.at[i0_v], y0_v, sem0)
            pltpu.async_copy(y_hbm.at[i1_v], y1_v, sem1)

        for b in (0, 1):
            _load(b, b)

        def _pair(j2, carry):
            for b in (0, 1):
                i0_v, i1_v, w0_v, w1_v, y0_v, y1_v, o_v, sem0, sem1 = bufs[b]
                j = j2 * 2 + b
                pltpu.make_async_copy(y_hbm.at[i0_v], y0_v, sem0).wait()
                pltpu.make_async_copy(y_hbm.at[i1_v], y1_v, sem1).wait()

                def _row(r, carry2):
                    wb0 = w0_v[r, pl.ds(0, 16)]
                    wb1 = w1_v[r, pl.ds(0, 16)]

                    def _add(g, carry3):
                        s = pl.ds(g * 16, 16)
                        o_v[r, s] = wb0 * y0_v[r, s] + wb1 * y1_v[r, s]
                        return carry3
                    return lax.fori_loop(0, ng, _add, carry2, unroll=8)

                lax.fori_loop(0, CH, _row, 0)
                bb = pl.multiple_of(base + j * CH, CH)
                pltpu.sync_copy(o_v, out_hbm.at[pl.ds(bb, CH)])

                @pl.when(j + 2 < n_ch)
                def _next():
                    _load(j + 2, b)
            return carry

        lax.fori_loop(0, n_ch // 2, _pair, 0)

    return combine


def kernel(x, gate_W, gate_b, expert_W, expert_b):
    batch, seq, H = x.shape
    E = gate_W.shape[0]
    T = batch * seq
    P = 2 * T + E * G
    NT = P // G
    xs = x.reshape(T, H)

    logits, idx, wn = _router(xs, gate_W, gate_b)

    e_all = idx.reshape(2 * T)
    oh = (e_all[:, None] == jnp.arange(E, dtype=jnp.int32)[None, :]).astype(jnp.int32)
    cum = jnp.cumsum(oh, axis=0)
    counts = cum[-1]
    rank = jnp.sum(cum * oh, axis=1) - 1
    padded = ((counts + G - 1) // G) * G
    starts = jnp.concatenate([jnp.zeros((1,), jnp.int32),
                              jnp.cumsum(padded)])[:E]
    dest = starts[e_all] + rank
    tile_start = jnp.arange(NT, dtype=jnp.int32) * G
    ends = starts + padded
    te = jnp.minimum(jnp.sum(tile_start[:, None] >= ends[None, :], axis=1),
                     E - 1).astype(jnp.int32)
    dpair = dest.reshape(T, 2)
    d0 = dpair[:, 0]
    d1 = dpair[:, 1]
    w0x = jnp.broadcast_to(wn[:, 0:1], (T, 16))
    w1x = jnp.broadcast_to(wn[:, 1:2], (T, 16))

    A = _make_dispatch(T, H, P)(xs, d0, d1)

    Y = _gmm(te, A, expert_W, expert_b.reshape(E, 1, H), P, H, E)

    out = _make_combine(T, H, P)(Y, d0, d1, w0x, w1x)
    return out.reshape(batch, seq, H), logits


# speedup vs baseline: 1.2915x; 1.2915x over previous
import functools

import jax
import jax.numpy as jnp
from jax import lax
from jax.experimental import pallas as pl
from jax.experimental.pallas import tpu as pltpu
from jax.experimental.pallas import tpu_sc as plsc

TOP_K = 2
G = 256


def _router_body(x_ref, gw_ref, gb_ref, logits_ref, idx_ref, wn_ref):
    x = x_ref[...]
    logits = lax.dot_general(
        x, gw_ref[...], (((1,), (1,)), ((), ())),
        preferred_element_type=jnp.float32) + gb_ref[...]
    logits_ref[...] = logits
    m1 = jnp.max(logits, axis=-1, keepdims=True)
    a1 = jnp.argmax(logits, axis=-1)
    cols = lax.broadcasted_iota(jnp.int32, logits.shape, 1)
    logits2 = jnp.where(cols == a1[:, None], -jnp.inf, logits)
    m2 = jnp.max(logits2, axis=-1, keepdims=True)
    a2 = jnp.argmax(logits2, axis=-1)
    w1 = 1.0 / (1.0 + jnp.exp(m2 - m1))
    idx_ref[...] = jnp.concatenate(
        [a1[:, None], a2[:, None]], axis=1).astype(jnp.int32)
    wn_ref[...] = jnp.concatenate([w1, 1.0 - w1], axis=1)


def _router(xs, gate_W, gate_b):
    T, H = xs.shape
    E = gate_W.shape[0]
    TM = 1024
    return pl.pallas_call(
        _router_body,
        grid=(T // TM,),
        in_specs=[
            pl.BlockSpec((TM, H), lambda i: (i, 0)),
            pl.BlockSpec((E, H), lambda i: (0, 0)),
            pl.BlockSpec((1, E), lambda i: (0, 0)),
        ],
        out_specs=[
            pl.BlockSpec((TM, E), lambda i: (i, 0)),
            pl.BlockSpec((TM, TOP_K), lambda i: (i, 0)),
            pl.BlockSpec((TM, TOP_K), lambda i: (i, 0)),
        ],
        out_shape=[
            jax.ShapeDtypeStruct((T, E), jnp.float32),
            jax.ShapeDtypeStruct((T, TOP_K), jnp.int32),
            jax.ShapeDtypeStruct((T, TOP_K), jnp.float32),
        ],
    )(xs, gate_W, gate_b.reshape(1, E))


def _make_dispatch(T, H, P):
    info = plsc.get_sparse_core_info()
    NC, NS = info.num_cores, info.num_subcores
    NW = NC * NS
    tok_w = T // NW
    CH = 16
    n_ch = tok_w // CH
    mesh = plsc.VectorSubcoreMesh(core_axis_name="c", subcore_axis_name="s")

    @functools.partial(
        pl.kernel, mesh=mesh,
        out_type=jax.ShapeDtypeStruct((P, H), jnp.float32),
        scratch_types=[
            pltpu.VMEM((CH, H), jnp.float32),
            pltpu.VMEM((CH, H), jnp.float32),
            pltpu.VMEM((CH,), jnp.int32),
            pltpu.VMEM((CH,), jnp.int32),
            pltpu.VMEM((CH,), jnp.int32),
            pltpu.VMEM((CH,), jnp.int32),
            pltpu.SemaphoreType.DMA,
            pltpu.SemaphoreType.DMA,
            pltpu.SemaphoreType.DMA,
            pltpu.SemaphoreType.DMA,
        ],
    )
    def dispatch(x_hbm, d0_hbm, d1_hbm, a_hbm,
                 xva, xvb, i0a, i0b, i1a, i1b, s0a, s0b, s1a, s1b):
        wid = lax.axis_index("s") * NC + lax.axis_index("c")
        base = pl.multiple_of(wid * tok_w, CH)
        bufs = ((xva, i0a, i1a, s0a, s1a), (xvb, i0b, i1b, s0b, s1b))

        def _pair(j2, carry):
            for b in (0, 1):
                xv, i0_v, i1_v, sem0, sem1 = bufs[b]
                j = j2 * 2 + b

                @pl.when(j2 > 0)
                def _drain():
                    pltpu.make_async_copy(xv, a_hbm.at[i0_v], sem0).wait()
                    pltpu.make_async_copy(xv, a_hbm.at[i1_v], sem1).wait()

                bb = pl.multiple_of(base + j * CH, CH)
                pltpu.sync_copy(x_hbm.at[pl.ds(bb, CH)], xv)
                pltpu.sync_copy(d0_hbm.at[pl.ds(bb, CH)], i0_v)
                pltpu.sync_copy(d1_hbm.at[pl.ds(bb, CH)], i1_v)
                pltpu.async_copy(xv, a_hbm.at[i0_v], sem0)
                pltpu.async_copy(xv, a_hbm.at[i1_v], sem1)
            return carry

        lax.fori_loop(0, n_ch // 2, _pair, 0)
        for b in (0, 1):
            xv, i0_v, i1_v, sem0, sem1 = bufs[b]
            pltpu.make_async_copy(xv, a_hbm.at[i0_v], sem0).wait()
            pltpu.make_async_copy(xv, a_hbm.at[i1_v], sem1).wait()

    return dispatch


def _gmm_body(te_ref, a_ref, w_ref, eb_ref, y_ref):
    acc = lax.dot_general(
        a_ref[...], w_ref[0], (((1,), (1,)), ((), ())),
        preferred_element_type=jnp.float32)
    y_ref[...] = acc + eb_ref[0]


def _gmm(te, A, eW, eb3, P, H, E):
    NT = P // G
    grid_spec = pltpu.PrefetchScalarGridSpec(
        num_scalar_prefetch=1,
        grid=(NT,),
        in_specs=[
            pl.BlockSpec((G, H), lambda i, te: (i, 0)),
            pl.BlockSpec((1, H, H), lambda i, te: (te[i], 0, 0)),
            pl.BlockSpec((1, 1, H), lambda i, te: (te[i], 0, 0)),
        ],
        out_specs=pl.BlockSpec((G, H), lambda i, te: (i, 0)),
    )
    return pl.pallas_call(
        _gmm_body,
        grid_spec=grid_spec,
        out_shape=jax.ShapeDtypeStruct((P, H), jnp.float32),
    )(te, A, eW, eb3)


def _make_combine(T, H, P):
    info = plsc.get_sparse_core_info()
    NC, NS = info.num_cores, info.num_subcores
    NW = NC * NS
    tok_w = T // NW
    CH = 8
    n_ch = tok_w // CH
    mesh = plsc.VectorSubcoreMesh(core_axis_name="c", subcore_axis_name="s")

    buf = lambda *s: [pltpu.VMEM(s, jnp.float32) for _ in range(2)]

    @functools.partial(
        pl.kernel, mesh=mesh,
        out_type=jax.ShapeDtypeStruct((T, H), jnp.float32),
        scratch_types=(
            [pltpu.VMEM((CH,), jnp.int32) for _ in range(4)]
            + buf(CH, 16) + buf(CH, 16) + buf(CH, H) + buf(CH, H) + buf(CH, H)
            + [pltpu.SemaphoreType.DMA for _ in range(4)]
        ),
    )
    def combine(y_hbm, d0_hbm, d1_hbm, w0_hbm, w1_hbm, out_hbm,
                i0a, i0b, i1a, i1b, w0a, w0b, w1a, w1b,
                y0a, y0b, y1a, y1b, oa, ob, s0a, s0b, s1a, s1b):
        wid = lax.axis_index("s") * NC + lax.axis_index("c")
        base = pl.multiple_of(wid * tok_w, CH)
        ng = H // 16
        bufs = ((i0a, i1a, w0a, w1a, y0a, y1a, oa, s0a, s1a),
                (i0b, i1b, w0b, w1b, y0b, y1b, ob, s0b, s1b))

        def _load(j, b):
            i0_v, i1_v, w0_v, w1_v, y0_v, y1_v, o_v, sem0, sem1 = bufs[b]
            bb = pl.multiple_of(base + j * CH, CH)
            pltpu.sync_copy(d0_hbm.at[pl.ds(bb, CH)], i0_v)
            pltpu.sync_copy(d1_hbm.at[pl.ds(bb, CH)], i1_v)
            pltpu.sync_copy(w0_hbm.at[pl.ds(bb, CH)], w0_v)
            pltpu.sync_copy(w1_hbm.at[pl.ds(bb, CH)], w1_v)
            pltpu.async_copy(y_hbm.at[i0_v], y0_v, sem0)
            pltpu.async_copy(y_hbm.at[i1_v], y1_v, sem1)

        for b in (0, 1):
            _load(b, b)

        def _pair(j2, carry):
            for b in (0, 1):
                i0_v, i1_v, w0_v, w1_v, y0_v, y1_v, o_v, sem0, sem1 = bufs[b]
                j = j2 * 2 + b
                pltpu.make_async_copy(y_hbm.at[i0_v], y0_v, sem0).wait()
                pltpu.make_async_copy(y_hbm.at[i1_v], y1_v, sem1).wait()

                def _row(r, carry2):
                    wb0 = w0_v[r, pl.ds(0, 16)]
                    wb1 = w1_v[r, pl.ds(0, 16)]

                    def _add(g, carry3):
                        s = pl.ds(g * 16, 16)
                        o_v[r, s] = wb0 * y0_v[r, s] + wb1 * y1_v[r, s]
                        return carry3
                    return lax.fori_loop(0, ng, _add, carry2, unroll=4)

                lax.fori_loop(0, CH, _row, 0)
                bb = pl.multiple_of(base + j * CH, CH)
                pltpu.sync_copy(o_v, out_hbm.at[pl.ds(bb, CH)])

                @pl.when(j + 2 < n_ch)
                def _next():
                    _load(j + 2, b)
            return carry

        lax.fori_loop(0, n_ch // 2, _pair, 0)

    return combine


def kernel(x, gate_W, gate_b, expert_W, expert_b):
    batch, seq, H = x.shape
    E = gate_W.shape[0]
    T = batch * seq
    P = 2 * T + E * G
    NT = P // G
    xs = x.reshape(T, H)

    logits, idx, wn = _router(xs, gate_W, gate_b)

    e_all = idx.reshape(2 * T)
    oh = (e_all[:, None] == jnp.arange(E, dtype=jnp.int32)[None, :]).astype(jnp.int32)
    cum = jnp.cumsum(oh, axis=0)
    counts = cum[-1]
    rank = jnp.sum(cum * oh, axis=1) - 1
    padded = ((counts + G - 1) // G) * G
    starts = jnp.concatenate([jnp.zeros((1,), jnp.int32),
                              jnp.cumsum(padded)])[:E]
    dest = starts[e_all] + rank
    tile_start = jnp.arange(NT, dtype=jnp.int32) * G
    ends = starts + padded
    te = jnp.minimum(jnp.sum(tile_start[:, None] >= ends[None, :], axis=1),
                     E - 1).astype(jnp.int32)
    dpair = dest.reshape(T, 2)
    d0 = dpair[:, 0]
    d1 = dpair[:, 1]
    w0x = jnp.broadcast_to(wn[:, 0:1], (T, 16))
    w1x = jnp.broadcast_to(wn[:, 1:2], (T, 16))

    A = _make_dispatch(T, H, P)(xs, d0, d1)

    Y = _gmm(te, A, expert_W, expert_b.reshape(E, 1, H), P, H, E)

    out = _make_combine(T, H, P)(Y, d0, d1, w0x, w1x)
    return out.reshape(batch, seq, H), logits


# speedup vs baseline: 1.3059x; 1.0112x over previous
import functools

import jax
import jax.numpy as jnp
from jax import lax
from jax.experimental import pallas as pl
from jax.experimental.pallas import tpu as pltpu
from jax.experimental.pallas import tpu_sc as plsc

TOP_K = 2
G = 256


def _router_body(x_ref, gw_ref, gb_ref,
                 logits_ref, idx_ref, wn_ref, rank_ref, counts_ref,
                 run_ref, *, nsteps):
    i = pl.program_id(0)

    @pl.when(i == 0)
    def _init():
        run_ref[...] = jnp.zeros_like(run_ref)

    x = x_ref[...]
    logits = lax.dot_general(
        x, gw_ref[...], (((1,), (1,)), ((), ())),
        preferred_element_type=jnp.float32) + gb_ref[...]
    logits_ref[...] = logits
    m1 = jnp.max(logits, axis=-1, keepdims=True)
    a1 = jnp.argmax(logits, axis=-1)
    cols = lax.broadcasted_iota(jnp.int32, logits.shape, 1)
    logits2 = jnp.where(cols == a1[:, None], -jnp.inf, logits)
    m2 = jnp.max(logits2, axis=-1, keepdims=True)
    a2 = jnp.argmax(logits2, axis=-1)
    w1 = 1.0 / (1.0 + jnp.exp(m2 - m1))
    idx_ref[...] = jnp.concatenate(
        [a1[:, None], a2[:, None]], axis=1).astype(jnp.int32)
    wn_ref[...] = jnp.concatenate([w1, 1.0 - w1], axis=1)

    oh1 = (cols == a1[:, None]).astype(jnp.float32)
    oh2 = (cols == a2[:, None]).astype(jnp.float32)
    oh = oh1 + oh2
    TM = oh.shape[0]
    rr = lax.broadcasted_iota(jnp.int32, (TM, TM), 0)
    cc = lax.broadcasted_iota(jnp.int32, (TM, TM), 1)
    tri = (rr > cc).astype(jnp.float32)
    c_before = lax.dot_general(
        tri, oh, (((1,), (0,)), ((), ())),
        preferred_element_type=jnp.float32)
    run = run_ref[...]
    rank0 = jnp.sum(oh1 * (run + c_before), axis=1)
    rank1 = jnp.sum(oh2 * (run + c_before + oh1), axis=1)
    rank_ref[...] = jnp.concatenate(
        [rank0[:, None], rank1[:, None]], axis=1).astype(jnp.int32)
    run_ref[...] = run + jnp.sum(oh, axis=0, keepdims=True)

    @pl.when(i == nsteps - 1)
    def _fin():
        counts_ref[...] = run_ref[...]


def _router(xs, gate_W, gate_b):
    T, H = xs.shape
    E = gate_W.shape[0]
    TM = 1024
    nsteps = T // TM
    return pl.pallas_call(
        functools.partial(_router_body, nsteps=nsteps),
        grid=(nsteps,),
        in_specs=[
            pl.BlockSpec((TM, H), lambda i: (i, 0)),
            pl.BlockSpec((E, H), lambda i: (0, 0)),
            pl.BlockSpec((1, E), lambda i: (0, 0)),
        ],
        out_specs=[
            pl.BlockSpec((TM, E), lambda i: (i, 0)),
            pl.BlockSpec((TM, TOP_K), lambda i: (i, 0)),
            pl.BlockSpec((TM, TOP_K), lambda i: (i, 0)),
            pl.BlockSpec((TM, TOP_K), lambda i: (i, 0)),
            pl.BlockSpec((1, E), lambda i: (0, 0)),
        ],
        out_shape=[
            jax.ShapeDtypeStruct((T, E), jnp.float32),
            jax.ShapeDtypeStruct((T, TOP_K), jnp.int32),
            jax.ShapeDtypeStruct((T, TOP_K), jnp.float32),
            jax.ShapeDtypeStruct((T, TOP_K), jnp.int32),
            jax.ShapeDtypeStruct((1, E), jnp.float32),
        ],
        scratch_shapes=[pltpu.VMEM((1, E), jnp.float32)],
    )(xs, gate_W, gate_b.reshape(1, E))


def _dest_body(idx_ref, rank_ref, wn_ref, counts_ref,
               d0_ref, d1_ref, w0x_ref, w1x_ref, te_ref, *, E, G, NT):
    counts = counts_ref[...]
    padded = jnp.ceil(counts / G) * G
    ee = lax.broadcasted_iota(jnp.int32, (E, E), 0)
    ff = lax.broadcasted_iota(jnp.int32, (E, E), 1)
    lt = (ee < ff).astype(jnp.float32)
    starts = lax.dot_general(
        padded, lt, (((1,), (0,)), ((), ())),
        preferred_element_type=jnp.float32)
    idx = idx_ref[...]
    cols0 = lax.broadcasted_iota(jnp.int32, (idx.shape[0], E), 1)
    oh1 = (cols0 == idx[:, 0][:, None]).astype(jnp.float32)
    oh2 = (cols0 == idx[:, 1][:, None]).astype(jnp.float32)
    s0 = jnp.sum(oh1 * starts, axis=1)
    s1 = jnp.sum(oh2 * starts, axis=1)
    rank = rank_ref[...]
    d0_ref[...] = (s0.astype(jnp.int32) + rank[:, 0])[:, None]
    d1_ref[...] = (s1.astype(jnp.int32) + rank[:, 1])[:, None]
    wn = wn_ref[...]
    w0x_ref[...] = jnp.broadcast_to(wn[:, 0][:, None], w0x_ref.shape)
    w1x_ref[...] = jnp.broadcast_to(wn[:, 1][:, None], w1x_ref.shape)
    ends = starts + padded
    tiles = lax.broadcasted_iota(jnp.int32, (NT, E), 0).astype(jnp.float32) * G
    te = jnp.sum((tiles >= ends).astype(jnp.int32), axis=1)
    te_ref[...] = jnp.minimum(te, E - 1).astype(jnp.int32)[None, :]


def _dest_builder(idx, rank, wn, counts, E, NT):
    T = idx.shape[0]
    return pl.pallas_call(
        functools.partial(_dest_body, E=E, G=G, NT=NT),
        grid=(1,),
        in_specs=[
            pl.BlockSpec((T, TOP_K), lambda i: (0, 0)),
            pl.BlockSpec((T, TOP_K), lambda i: (0, 0)),
            pl.BlockSpec((T, TOP_K), lambda i: (0, 0)),
            pl.BlockSpec((1, E), lambda i: (0, 0)),
        ],
        out_specs=[
            pl.BlockSpec((T, 1), lambda i: (0, 0)),
            pl.BlockSpec((T, 1), lambda i: (0, 0)),
            pl.BlockSpec((T, 16), lambda i: (0, 0)),
            pl.BlockSpec((T, 16), lambda i: (0, 0)),
            pl.BlockSpec((1, NT), lambda i: (0, 0)),
        ],
        out_shape=[
            jax.ShapeDtypeStruct((T, 1), jnp.int32),
            jax.ShapeDtypeStruct((T, 1), jnp.int32),
            jax.ShapeDtypeStruct((T, 16), jnp.float32),
            jax.ShapeDtypeStruct((T, 16), jnp.float32),
            jax.ShapeDtypeStruct((1, NT), jnp.int32),
        ],
    )(idx, rank, wn, counts)


def _make_dispatch(T, H, P):
    info = plsc.get_sparse_core_info()
    NC, NS = info.num_cores, info.num_subcores
    NW = NC * NS
    tok_w = T // NW
    CH = 16
    n_ch = tok_w // CH
    mesh = plsc.VectorSubcoreMesh(core_axis_name="c", subcore_axis_name="s")

    @functools.partial(
        pl.kernel, mesh=mesh,
        out_type=jax.ShapeDtypeStruct((P, H), jnp.float32),
        scratch_types=[
            pltpu.VMEM((CH, H), jnp.float32),
            pltpu.VMEM((CH, H), jnp.float32),
            pltpu.VMEM((CH,), jnp.int32),
            pltpu.VMEM((CH,), jnp.int32),
            pltpu.VMEM((CH,), jnp.int32),
            pltpu.VMEM((CH,), jnp.int32),
            pltpu.SemaphoreType.DMA,
            pltpu.SemaphoreType.DMA,
            pltpu.SemaphoreType.DMA,
            pltpu.SemaphoreType.DMA,
        ],
    )
    def dispatch(x_hbm, d0_hbm, d1_hbm, a_hbm,
                 xva, xvb, i0a, i0b, i1a, i1b, s0a, s0b, s1a, s1b):
        wid = lax.axis_index("s") * NC + lax.axis_index("c")
        base = pl.multiple_of(wid * tok_w, CH)
        bufs = ((xva, i0a, i1a, s0a, s1a), (xvb, i0b, i1b, s0b, s1b))

        def _pair(j2, carry):
            for b in (0, 1):
                xv, i0_v, i1_v, sem0, sem1 = bufs[b]
                j = j2 * 2 + b

                @pl.when(j2 > 0)
                def _drain():
                    pltpu.make_async_copy(xv, a_hbm.at[i0_v], sem0).wait()
                    pltpu.make_async_copy(xv, a_hbm.at[i1_v], sem1).wait()

                bb = pl.multiple_of(base + j * CH, CH)
                pltpu.sync_copy(x_hbm.at[pl.ds(bb, CH)], xv)
                pltpu.sync_copy(d0_hbm.at[pl.ds(bb, CH)], i0_v)
                pltpu.sync_copy(d1_hbm.at[pl.ds(bb, CH)], i1_v)
                pltpu.async_copy(xv, a_hbm.at[i0_v], sem0)
                pltpu.async_copy(xv, a_hbm.at[i1_v], sem1)
            return carry

        lax.fori_loop(0, n_ch // 2, _pair, 0)
        for b in (0, 1):
            xv, i0_v, i1_v, sem0, sem1 = bufs[b]
            pltpu.make_async_copy(xv, a_hbm.at[i0_v], sem0).wait()
            pltpu.make_async_copy(xv, a_hbm.at[i1_v], sem1).wait()

    return dispatch


def _gmm_body(te_ref, a_ref, w_ref, eb_ref, y_ref):
    acc = lax.dot_general(
        a_ref[...], w_ref[0], (((1,), (1,)), ((), ())),
        preferred_element_type=jnp.float32)
    y_ref[...] = acc + eb_ref[0]


def _gmm(te, A, eW, eb3, P, H, E):
    NT = P // G
    grid_spec = pltpu.PrefetchScalarGridSpec(
        num_scalar_prefetch=1,
        grid=(NT,),
        in_specs=[
            pl.BlockSpec((G, H), lambda i, te: (i, 0)),
            pl.BlockSpec((1, H, H), lambda i, te: (te[i], 0, 0)),
            pl.BlockSpec((1, 1, H), lambda i, te: (te[i], 0, 0)),
        ],
        out_specs=pl.BlockSpec((G, H), lambda i, te: (i, 0)),
    )
    return pl.pallas_call(
        _gmm_body,
        grid_spec=grid_spec,
        out_shape=jax.ShapeDtypeStruct((P, H), jnp.float32),
    )(te, A, eW, eb3)


def _make_combine(T, H, P):
    info = plsc.get_sparse_core_info()
    NC, NS = info.num_cores, info.num_subcores
    NW = NC * NS
    tok_w = T // NW
    CH = 8
    n_ch = tok_w // CH
    mesh = plsc.VectorSubcoreMesh(core_axis_name="c", subcore_axis_name="s")

    buf = lambda *s: [pltpu.VMEM(s, jnp.float32) for _ in range(2)]

    @functools.partial(
        pl.kernel, mesh=mesh,
        out_type=jax.ShapeDtypeStruct((T, H), jnp.float32),
        scratch_types=(
            [pltpu.VMEM((CH,), jnp.int32) for _ in range(4)]
            + buf(CH, 16) + buf(CH, 16) + buf(CH, H) + buf(CH, H) + buf(CH, H)
            + [pltpu.SemaphoreType.DMA for _ in range(4)]
        ),
    )
    def combine(y_hbm, d0_hbm, d1_hbm, w0_hbm, w1_hbm, out_hbm,
                i0a, i0b, i1a, i1b, w0a, w0b, w1a, w1b,
                y0a, y0b, y1a, y1b, oa, ob, s0a, s0b, s1a, s1b):
        wid = lax.axis_index("s") * NC + lax.axis_index("c")
        base = pl.multiple_of(wid * tok_w, CH)
        ng = H // 16
        bufs = ((i0a, i1a, w0a, w1a, y0a, y1a, oa, s0a, s1a),
                (i0b, i1b, w0b, w1b, y0b, y1b, ob, s0b, s1b))

        def _load(j, b):
            i0_v, i1_v, w0_v, w1_v, y0_v, y1_v, o_v, sem0, sem1 = bufs[b]
            bb = pl.multiple_of(base + j * CH, CH)
            pltpu.sync_copy(d0_hbm.at[pl.ds(bb, CH)], i0_v)
            pltpu.sync_copy(d1_hbm.at[pl.ds(bb, CH)], i1_v)
            pltpu.sync_copy(w0_hbm.at[pl.ds(bb, CH)], w0_v)
            pltpu.sync_copy(w1_hbm.at[pl.ds(bb, CH)], w1_v)
            pltpu.async_copy(y_hbm.at[i0_v], y0_v, sem0)
            pltpu.async_copy(y_hbm.at[i1_v], y1_v, sem1)

        for b in (0, 1):
            _load(b, b)

        def _pair(j2, carry):
            for b in (0, 1):
                i0_v, i1_v, w0_v, w1_v, y0_v, y1_v, o_v, sem0, sem1 = bufs[b]
                j = j2 * 2 + b
                pltpu.make_async_copy(y_hbm.at[i0_v], y0_v, sem0).wait()
                pltpu.make_async_copy(y_hbm.at[i1_v], y1_v, sem1).wait()

                def _row(r, carry2):
                    wb0 = w0_v[r, pl.ds(0, 16)]
                    wb1 = w1_v[r, pl.ds(0, 16)]

                    def _add(g, carry3):
                        s = pl.ds(g * 16, 16)
                        o_v[r, s] = wb0 * y0_v[r, s] + wb1 * y1_v[r, s]
                        return carry3
                    return lax.fori_loop(0, ng, _add, carry2, unroll=4)

                lax.fori_loop(0, CH, _row, 0)
                bb = pl.multiple_of(base + j * CH, CH)
                pltpu.sync_copy(o_v, out_hbm.at[pl.ds(bb, CH)])

                @pl.when(j + 2 < n_ch)
                def _next():
                    _load(j + 2, b)
            return carry

        lax.fori_loop(0, n_ch // 2, _pair, 0)

    return combine


def kernel(x, gate_W, gate_b, expert_W, expert_b):
    batch, seq, H = x.shape
    E = gate_W.shape[0]
    T = batch * seq
    P = 2 * T + E * G
    NT = P // G
    xs = x.reshape(T, H)

    logits, idx, wn, rank, counts = _router(xs, gate_W, gate_b)
    d0, d1, w0x, w1x, te2 = _dest_builder(idx, rank, wn, counts, E, NT)
    d0 = d0.reshape(T)
    d1 = d1.reshape(T)
    te = te2.reshape(NT)

    A = _make_dispatch(T, H, P)(xs, d0, d1)

    Y = _gmm(te, A, expert_W, expert_b.reshape(E, 1, H), P, H, E)

    out = _make_combine(T, H, P)(Y, d0, d1, w0x, w1x)
    return out.reshape(batch, seq, H), logits
